# baseline (speedup 1.0000x reference)
import functools

import jax
import jax.numpy as jnp
from jax import lax
from jax.experimental import pallas as pl
from jax.experimental.pallas import tpu as pltpu, tpu_sc as plsc

N_NODES = 10000
N_EDGES = 320000
D_IN = 128
D_OUT = 128

DH = D_IN // 2
NS = 16
N_PAD = 10240
ROWS_PT = N_PAD // NS
EB = 80
EROWS = N_EDGES // EB
EROWS_PT = EROWS // NS
CH = 25
NCH = EROWS_PT // CH
RCH = 16
NRCH = ROWS_PT // RCH
DROWS = N_PAD // 16
FROWS_PT = N_NODES // NS
IROWS = N_PAD // 2048


def _sc_aggregate(feat, src3, dst3):
    mesh = plsc.VectorSubcoreMesh(core_axis_name="c", subcore_axis_name="s")

    @functools.partial(
        pl.kernel,
        out_type=(
            jax.ShapeDtypeStruct((N_PAD, D_IN), jnp.float32),
            jax.ShapeDtypeStruct((DROWS, 16), jnp.float32),
        ),
        mesh=mesh,
        scratch_types=[
            pltpu.VMEM_SHARED((N_PAD, DH), jnp.float32),
            pltpu.VMEM_SHARED((N_PAD, DH), jnp.float32),
            pltpu.VMEM_SHARED((DROWS, 16), jnp.float32),
            pltpu.VMEM((2, CH, EB), jnp.int32),
            pltpu.VMEM((2, CH, EB), jnp.int32),
            pltpu.VMEM((4, EB, DH), jnp.float32),
            pltpu.VMEM((RCH, DH), jnp.float32),
            pltpu.VMEM((N_PAD,), jnp.float32),
            pltpu.VMEM((32, 16), jnp.float32),
            pltpu.VMEM((20, 32), jnp.int32),
        ] + [pltpu.SemaphoreType.DMA] * 6,
        compiler_params=pltpu.CompilerParams(
            use_tc_tiling_on_sc=False, needs_layout_passes=False),
    )
    def k(feat_hbm, src_hbm, dst_hbm, agg_hbm, deg_hbm,
          feat_sp, acc_sp, deg_sp, src_v, dst_v, rows4, zero_v, deg_l,
          deg_l2, idx64, *sems):
        c = lax.axis_index("c")
        s = lax.axis_index("s")
        r0 = s * ROWS_PT
        gsems = sems[:2]
        ssems = sems[2:4]
        isems = sems[4:]

        f0 = s * FROWS_PT
        pltpu.sync_copy(feat_hbm.at[pl.ds(f0, FROWS_PT), pl.ds(c * DH, DH)],
                        feat_sp.at[pl.ds(f0, FROWS_PT)])

        zf = jnp.zeros((16,), jnp.float32)

        def zero_stage(i, _):
            for j in range(DH // 16):
                zero_v[i, pl.ds(16 * j, 16)] = zf
            return 0

        lax.fori_loop(0, RCH, zero_stage, 0)

        def zero_hist(i, _):
            deg_l[pl.ds(16 * i, 16)] = zf
            return 0

        lax.fori_loop(0, DROWS, zero_hist, 0)

        def zero_hist2(i, _):
            deg_l2[i] = zf
            return 0

        lax.fori_loop(0, 32, zero_hist2, 0)

        iot = lax.iota(jnp.int32, 16)

        def fill_iota(q, _):
            for t in range(2):
                idx64[q, pl.ds(16 * t, 16)] = iot + q * 32 + 16 * t
            return 0

        lax.fori_loop(0, 20, fill_iota, 0)

        def zero_copy(kk, _):
            pltpu.sync_copy(zero_v, acc_sp.at[pl.ds(r0 + kk * RCH, RCH)])
            return 0

        lax.fori_loop(0, NRCH, zero_copy, 0)
        pltpu.sync_copy(deg_l2.at[pl.ds(0, DROWS // NS)],
                        deg_sp.at[pl.ds(s * (DROWS // NS), DROWS // NS)])

        plsc.subcore_barrier()

        of = jnp.ones((16,), jnp.float32)

        pltpu.sync_copy(src_hbm.at[s, pl.ds(0, CH)], src_v.at[0])
        pltpu.sync_copy(dst_hbm.at[s, pl.ds(0, CH)], dst_v.at[0])
        pltpu.async_copy(src_hbm.at[s, pl.ds(CH, CH)], src_v.at[1], isems[0])
        pltpu.async_copy(dst_hbm.at[s, pl.ds(CH, CH)], dst_v.at[1], isems[1])

        def idx_at(ref, j):
            return ref.at[lax.rem(lax.div(j, CH), 2), lax.rem(j, CH)]

        pltpu.async_copy(feat_sp.at[idx_at(src_v, 0)], rows4.at[0], gsems[0])
        pltpu.async_copy(feat_sp.at[idx_at(src_v, 1)], rows4.at[1], gsems[1])

        def pair_body(jj, _):
            for p in range(2):
                j = 2 * jj + p
                gs = gsems[p]
                ss = ssems[p]
                b = lax.rem(j, 4)
                bn = lax.rem(j + 2, 4)
                pltpu.make_async_copy(
                    feat_sp.at[idx_at(src_v, j)], rows4.at[b], gs).wait()

                @pl.when(j >= 2)
                def _():
                    pltpu.make_async_copy(
                        rows4.at[bn], acc_sp.at[idx_at(dst_v, j - 2)],
                        ss).wait()

                @pl.when((lax.rem(j + 2, CH) == 0) & (j + 2 < EROWS_PT))
                def _():
                    nsl = lax.rem(lax.div(j + 2, CH), 2)
                    pltpu.make_async_copy(
                        src_hbm.at[s, pl.ds(j + 2, CH)], src_v.at[nsl],
                        isems[0]).wait()
                    pltpu.make_async_copy(
                        dst_hbm.at[s, pl.ds(j + 2, CH)], dst_v.at[nsl],
                        isems[1]).wait()

                @pl.when((lax.rem(j, CH) == 2) & (j - 2 + CH < EROWS_PT)
                         & (j > CH))
                def _():
                    csl = lax.rem(lax.div(j, CH), 2)
                    e2 = j - 2 + CH
                    pltpu.async_copy(
                        src_hbm.at[s, pl.ds(e2, CH)], src_v.at[1 - csl],
                        isems[0])
                    pltpu.async_copy(
                        dst_hbm.at[s, pl.ds(e2, CH)], dst_v.at[1 - csl],
                        isems[1])

                @pl.when(j + 2 < EROWS_PT)
                def _():
                    pltpu.async_copy(
                        feat_sp.at[idx_at(src_v, j + 2)], rows4.at[bn], gs)

                pltpu.async_copy(
                    rows4.at[b], acc_sp.at[idx_at(dst_v, j)], ss, add=True)

                sl = lax.rem(lax.div(j, CH), 2)
                jr = lax.rem(j, CH)
                for t in range(EB // 16):
                    idx = dst_v[sl, jr, pl.ds(16 * t, 16)]
                    plsc.addupdate_scatter(deg_l, [idx], of)
            return 0

        lax.fori_loop(0, EROWS_PT // 2, pair_body, 0)
        for dd in (EROWS_PT - 2, EROWS_PT - 1):
            pltpu.make_async_copy(
                rows4.at[dd % 4], acc_sp.at[idx_at(dst_v, dd)],
                ssems[dd % 2]).wait()

        def merge_body(kk, _):
            def repack(q, _):
                deg_l2[q] = deg_l[pl.ds(512 * kk + 16 * q, 16)]
                return 0

            lax.fori_loop(0, 32, repack, 0)
            pltpu.sync_copy(deg_l2, deg_sp.at[idx64.at[kk]], add=True)
            return 0

        lax.fori_loop(0, 20, merge_body, 0)

        plsc.subcore_barrier()

        pltpu.sync_copy(acc_sp.at[pl.ds(r0, ROWS_PT)],
                        agg_hbm.at[pl.ds(r0, ROWS_PT), pl.ds(c * DH, DH)])

        @pl.when(c == 0)
        def _():
            pltpu.sync_copy(deg_sp.at[pl.ds(s * (DROWS // NS), DROWS // NS)],
                            deg_hbm.at[pl.ds(s * (DROWS // NS), DROWS // NS)])

    return k(feat, src3, dst3)


def _tc_body(feat_ref, agg_ref, deg_ref, w1_ref, w2_ref, b_ref, o_ref):
    dinv = 1.0 / jnp.maximum(deg_ref[...], 1.0)
    o_ref[...] = (
        jnp.dot(feat_ref[...], w1_ref[...], preferred_element_type=jnp.float32)
        + jnp.dot(agg_ref[...] * dinv, w2_ref[...],
                  preferred_element_type=jnp.float32)
        + b_ref[...]
    )


RB = 1000


def _tc_linear(feat, agg, deg, w1t, w2t, b2d):
    grid = (N_NODES // RB,)
    return pl.pallas_call(
        _tc_body,
        grid=grid,
        in_specs=[
            pl.BlockSpec((RB, D_IN), lambda i: (i, 0)),
            pl.BlockSpec((RB, D_IN), lambda i: (i, 0)),
            pl.BlockSpec((RB, 1), lambda i: (i, 0)),
            pl.BlockSpec((D_IN, D_OUT), lambda i: (0, 0)),
            pl.BlockSpec((D_IN, D_OUT), lambda i: (0, 0)),
            pl.BlockSpec((1, D_OUT), lambda i: (0, 0)),
        ],
        out_specs=pl.BlockSpec((RB, D_OUT), lambda i: (i, 0)),
        out_shape=jax.ShapeDtypeStruct((N_NODES, D_OUT), jnp.float32),
    )(feat, agg, deg, w1t, w2t, b2d)


def kernel(feat, edge_index, W, b):
    src3 = edge_index[0].astype(jnp.int32).reshape(NS, EROWS_PT, EB)
    dst3 = edge_index[1].astype(jnp.int32).reshape(NS, EROWS_PT, EB)
    agg, deg = _sc_aggregate(feat, src3, dst3)
    deg2d = deg.reshape(N_PAD, 1)[:N_NODES]
    return _tc_linear(feat, agg, deg2d, W[:, :D_IN].T, W[:, D_IN:].T,
                      b.reshape(1, D_OUT))


# speedup vs baseline: 1.0037x; 1.0037x over previous
import functools

import jax
import jax.numpy as jnp
from jax import lax
from jax.experimental import pallas as pl
from jax.experimental.pallas import tpu as pltpu, tpu_sc as plsc

N_NODES = 10000
N_EDGES = 320000
D_IN = 128
D_OUT = 128

DH = D_IN // 2
NS = 16
N_PAD = 10240
ROWS_PT = N_PAD // NS
EB = 80
EROWS = N_EDGES // EB
EROWS_PT = EROWS // NS
CH = 25
NCH = EROWS_PT // CH
RCH = 16
NRCH = ROWS_PT // RCH
DROWS = N_PAD // 16
FROWS_PT = N_NODES // NS
IROWS = N_PAD // 2048


def _sc_aggregate(feat, src3, dst3):
    mesh = plsc.VectorSubcoreMesh(core_axis_name="c", subcore_axis_name="s")

    @functools.partial(
        pl.kernel,
        out_type=(
            jax.ShapeDtypeStruct((N_PAD, D_IN), jnp.float32),
            jax.ShapeDtypeStruct((DROWS, 16), jnp.float32),
        ),
        mesh=mesh,
        scratch_types=[
            pltpu.VMEM_SHARED((N_PAD, DH), jnp.float32),
            pltpu.VMEM_SHARED((N_PAD, DH), jnp.float32),
            pltpu.VMEM_SHARED((DROWS, 16), jnp.float32),
            pltpu.VMEM((2, CH, EB), jnp.int32),
            pltpu.VMEM((2, CH, EB), jnp.int32),
            pltpu.VMEM((4, EB, DH), jnp.float32),
            pltpu.VMEM((RCH, DH), jnp.float32),
            pltpu.VMEM((N_PAD,), jnp.float32),
            pltpu.VMEM((32, 16), jnp.float32),
            pltpu.VMEM((20, 32), jnp.int32),
        ] + [pltpu.SemaphoreType.DMA] * 6,
        compiler_params=pltpu.CompilerParams(
            use_tc_tiling_on_sc=False, needs_layout_passes=False),
    )
    def k(feat_hbm, src_hbm, dst_hbm, agg_hbm, deg_hbm,
          feat_sp, acc_sp, deg_sp, src_v, dst_v, rows4, zero_v, deg_l,
          deg_l2, idx64, *sems):
        c = lax.axis_index("c")
        s = lax.axis_index("s")
        r0 = s * ROWS_PT
        gsems = sems[:2]
        ssems = sems[2:4]
        isems = sems[4:]

        f0 = s * FROWS_PT
        pltpu.sync_copy(feat_hbm.at[pl.ds(f0, FROWS_PT), pl.ds(c * DH, DH)],
                        feat_sp.at[pl.ds(f0, FROWS_PT)])

        zf = jnp.zeros((16,), jnp.float32)

        def zero_stage(i, _):
            for j in range(DH // 16):
                zero_v[i, pl.ds(16 * j, 16)] = zf
            return 0

        lax.fori_loop(0, RCH, zero_stage, 0)

        def zero_hist(i, _):
            deg_l[pl.ds(16 * i, 16)] = zf
            return 0

        lax.fori_loop(0, DROWS, zero_hist, 0)

        def zero_hist2(i, _):
            deg_l2[i] = zf
            return 0

        lax.fori_loop(0, 32, zero_hist2, 0)

        iot = lax.iota(jnp.int32, 16)

        def fill_iota(q, _):
            for t in range(2):
                idx64[q, pl.ds(16 * t, 16)] = iot + q * 32 + 16 * t
            return 0

        lax.fori_loop(0, 20, fill_iota, 0)

        def zero_copy(kk, _):
            pltpu.sync_copy(zero_v, acc_sp.at[pl.ds(r0 + kk * RCH, RCH)])
            return 0

        lax.fori_loop(0, NRCH, zero_copy, 0)
        pltpu.sync_copy(deg_l2.at[pl.ds(0, DROWS // NS)],
                        deg_sp.at[pl.ds(s * (DROWS // NS), DROWS // NS)])

        plsc.subcore_barrier()

        of = jnp.ones((16,), jnp.float32)

        pltpu.sync_copy(src_hbm.at[s, pl.ds(0, CH)], src_v.at[0])
        pltpu.sync_copy(dst_hbm.at[s, pl.ds(0, CH)], dst_v.at[0])
        pltpu.async_copy(src_hbm.at[s, pl.ds(CH, CH)], src_v.at[1], isems[0])
        pltpu.async_copy(dst_hbm.at[s, pl.ds(CH, CH)], dst_v.at[1], isems[1])

        def idx_at(ref, j):
            return ref.at[lax.rem(lax.div(j, CH), 2), lax.rem(j, CH)]

        pltpu.async_copy(feat_sp.at[idx_at(src_v, 0)], rows4.at[0], gsems[0])
        pltpu.async_copy(feat_sp.at[idx_at(src_v, 1)], rows4.at[1], gsems[1])

        def pair_body(jj, _):
            for p in range(2):
                j = 2 * jj + p
                gs = gsems[p]
                ss = ssems[p]
                b = lax.rem(j, 4)
                bn = lax.rem(j + 2, 4)
                pltpu.make_async_copy(
                    feat_sp.at[idx_at(src_v, j)], rows4.at[b], gs).wait()

                @pl.when(j >= 2)
                def _():
                    pltpu.make_async_copy(
                        rows4.at[bn], acc_sp.at[idx_at(dst_v, j - 2)],
                        ss).wait()

                @pl.when((lax.rem(j + 2, CH) == 0) & (j + 2 < EROWS_PT))
                def _():
                    nsl = lax.rem(lax.div(j + 2, CH), 2)
                    pltpu.make_async_copy(
                        src_hbm.at[s, pl.ds(j + 2, CH)], src_v.at[nsl],
                        isems[0]).wait()
                    pltpu.make_async_copy(
                        dst_hbm.at[s, pl.ds(j + 2, CH)], dst_v.at[nsl],
                        isems[1]).wait()

                @pl.when((lax.rem(j, CH) == 2) & (j - 2 + CH < EROWS_PT)
                         & (j > CH))
                def _():
                    csl = lax.rem(lax.div(j, CH), 2)
                    e2 = j - 2 + CH
                    pltpu.async_copy(
                        src_hbm.at[s, pl.ds(e2, CH)], src_v.at[1 - csl],
                        isems[0])
                    pltpu.async_copy(
                        dst_hbm.at[s, pl.ds(e2, CH)], dst_v.at[1 - csl],
                        isems[1])

                @pl.when(j + 2 < EROWS_PT)
                def _():
                    pltpu.async_copy(
                        feat_sp.at[idx_at(src_v, j + 2)], rows4.at[bn], gs)

                pltpu.async_copy(
                    rows4.at[b], acc_sp.at[idx_at(dst_v, j)], ss, add=True)

                sl = lax.rem(lax.div(j, CH), 2)
                jr = lax.rem(j, CH)
                for t in range(EB // 16):
                    idx = dst_v[sl, jr, pl.ds(16 * t, 16)]
                    plsc.addupdate_scatter(deg_l, [idx], of)
            return 0

        lax.fori_loop(0, EROWS_PT // 2, pair_body, 0)
        for dd in (EROWS_PT - 2, EROWS_PT - 1):
            pltpu.make_async_copy(
                rows4.at[dd % 4], acc_sp.at[idx_at(dst_v, dd)],
                ssems[dd % 2]).wait()

        def merge_body(kk, _):
            def repack(q, _):
                deg_l2[q] = deg_l[pl.ds(512 * kk + 16 * q, 16)]
                return 0

            lax.fori_loop(0, 32, repack, 0)
            pltpu.sync_copy(deg_l2, deg_sp.at[idx64.at[kk]], add=True)
            return 0

        lax.fori_loop(0, 20, merge_body, 0)

        plsc.subcore_barrier()

        pltpu.sync_copy(acc_sp.at[pl.ds(r0, ROWS_PT)],
                        agg_hbm.at[pl.ds(r0, ROWS_PT), pl.ds(c * DH, DH)])

        @pl.when(c == 0)
        def _():
            pltpu.sync_copy(deg_sp.at[pl.ds(s * (DROWS // NS), DROWS // NS)],
                            deg_hbm.at[pl.ds(s * (DROWS // NS), DROWS // NS)])

    return k(feat, src3, dst3)


def _tc1_body(feat_ref, w1_ref, b_ref, o_ref):
    o_ref[...] = jnp.dot(feat_ref[...], w1_ref[...],
                         preferred_element_type=jnp.float32) + b_ref[...]


def _tc2_body(p1_ref, agg_ref, deg_ref, w2_ref, o_ref):
    dinv = 1.0 / jnp.maximum(deg_ref[...], 1.0)
    o_ref[...] = p1_ref[...] + jnp.dot(
        agg_ref[...] * dinv, w2_ref[...], preferred_element_type=jnp.float32)


RB = 1000


def _tc_linear1(feat, w1t, b2d):
    grid = (N_NODES // RB,)
    return pl.pallas_call(
        _tc1_body,
        grid=grid,
        in_specs=[
            pl.BlockSpec((RB, D_IN), lambda i: (i, 0)),
            pl.BlockSpec((D_IN, D_OUT), lambda i: (0, 0)),
            pl.BlockSpec((1, D_OUT), lambda i: (0, 0)),
        ],
        out_specs=pl.BlockSpec((RB, D_OUT), lambda i: (i, 0)),
        out_shape=jax.ShapeDtypeStruct((N_NODES, D_OUT), jnp.float32),
    )(feat, w1t, b2d)


def _tc_linear2(p1, agg, deg, w2t):
    grid = (N_NODES // RB,)
    return pl.pallas_call(
        _tc2_body,
        grid=grid,
        in_specs=[
            pl.BlockSpec((RB, D_OUT), lambda i: (i, 0)),
            pl.BlockSpec((RB, D_IN), lambda i: (i, 0)),
            pl.BlockSpec((RB, 1), lambda i: (i, 0)),
            pl.BlockSpec((D_IN, D_OUT), lambda i: (0, 0)),
        ],
        out_specs=pl.BlockSpec((RB, D_OUT), lambda i: (i, 0)),
        out_shape=jax.ShapeDtypeStruct((N_NODES, D_OUT), jnp.float32),
    )(p1, agg, deg, w2t)


def kernel(feat, edge_index, W, b):
    src3 = edge_index[0].astype(jnp.int32).reshape(NS, EROWS_PT, EB)
    dst3 = edge_index[1].astype(jnp.int32).reshape(NS, EROWS_PT, EB)
    agg, deg = _sc_aggregate(feat, src3, dst3)
    p1 = _tc_linear1(feat, W[:, :D_IN].T, b.reshape(1, D_OUT))
    deg2d = deg.reshape(N_PAD, 1)[:N_NODES]
    return _tc_linear2(p1, agg, deg2d, W[:, D_IN:].T)


# speedup vs baseline: 1.0044x; 1.0007x over previous
import functools

import jax
import jax.numpy as jnp
from jax import lax
from jax.experimental import pallas as pl
from jax.experimental.pallas import tpu as pltpu, tpu_sc as plsc

N_NODES = 10000
N_EDGES = 320000
D_IN = 128
D_OUT = 128

DH = D_IN // 2
NS = 16
N_PAD = 10240
ROWS_PT = N_PAD // NS
EB = 80
EROWS = N_EDGES // EB
EROWS_PT = EROWS // NS
CH = 25
NCH = EROWS_PT // CH
RCH = 16
NRCH = ROWS_PT // RCH
DROWS = N_PAD // 16
FROWS_PT = N_NODES // NS


def _sc_aggregate(feat, src3, dst3):
    mesh = plsc.VectorSubcoreMesh(core_axis_name="c", subcore_axis_name="s")

    @functools.partial(
        pl.kernel,
        out_type=(
            jax.ShapeDtypeStruct((N_PAD, D_IN), jnp.float32),
            jax.ShapeDtypeStruct((DROWS, 16), jnp.float32),
        ),
        mesh=mesh,
        scratch_types=[
            pltpu.VMEM_SHARED((N_PAD, DH), jnp.float32),
            pltpu.VMEM_SHARED((N_PAD, DH), jnp.float32),
            pltpu.VMEM_SHARED((DROWS, 16), jnp.float32),
            pltpu.VMEM((2, CH, EB), jnp.int32),
            pltpu.VMEM((2, CH, EB), jnp.int32),
            pltpu.VMEM((4, EB, DH), jnp.float32),
            pltpu.VMEM((RCH, DH), jnp.float32),
            pltpu.VMEM((N_PAD,), jnp.float32),
            pltpu.VMEM((32, 16), jnp.float32),
            pltpu.VMEM((20, 32), jnp.int32),
        ] + [pltpu.SemaphoreType.DMA] * 6,
        compiler_params=pltpu.CompilerParams(
            use_tc_tiling_on_sc=False, needs_layout_passes=False),
    )
    def k(feat_hbm, src_hbm, dst_hbm, agg_hbm, deg_hbm,
          feat_sp, acc_sp, deg_sp, src_v, dst_v, rows4, zero_v, deg_l,
          deg_l2, idx64, *sems):
        c = lax.axis_index("c")
        s = lax.axis_index("s")
        r0 = s * ROWS_PT
        gsems = sems[:2]
        ssems = sems[2:4]
        isems = sems[4:]

        f0 = s * FROWS_PT
        pltpu.sync_copy(feat_hbm.at[pl.ds(f0, FROWS_PT), pl.ds(c * DH, DH)],
                        feat_sp.at[pl.ds(f0, FROWS_PT)])

        zf = jnp.zeros((16,), jnp.float32)

        def zero_stage(i, _):
            for j in range(DH // 16):
                zero_v[i, pl.ds(16 * j, 16)] = zf
            return 0

        lax.fori_loop(0, RCH, zero_stage, 0)

        def zero_hist(i, _):
            deg_l[pl.ds(16 * i, 16)] = zf
            return 0

        lax.fori_loop(0, DROWS, zero_hist, 0)

        def zero_hist2(i, _):
            deg_l2[i] = zf
            return 0

        lax.fori_loop(0, 32, zero_hist2, 0)

        iot = lax.iota(jnp.int32, 16)

        def fill_iota(q, _):
            for t in range(2):
                idx64[q, pl.ds(16 * t, 16)] = iot + q * 32 + 16 * t
            return 0

        lax.fori_loop(0, 20, fill_iota, 0)

        def zero_copy(kk, _):
            pltpu.sync_copy(zero_v, acc_sp.at[pl.ds(r0 + kk * RCH, RCH)])
            return 0

        lax.fori_loop(0, NRCH, zero_copy, 0)
        pltpu.sync_copy(deg_l2.at[pl.ds(0, DROWS // NS)],
                        deg_sp.at[pl.ds(s * (DROWS // NS), DROWS // NS)])

        plsc.subcore_barrier()

        of = jnp.ones((16,), jnp.float32)

        pltpu.sync_copy(src_hbm.at[s, pl.ds(0, CH)], src_v.at[0])
        pltpu.sync_copy(dst_hbm.at[s, pl.ds(0, CH)], dst_v.at[0])
        pltpu.async_copy(src_hbm.at[s, pl.ds(CH, CH)], src_v.at[1], isems[0])
        pltpu.async_copy(dst_hbm.at[s, pl.ds(CH, CH)], dst_v.at[1], isems[1])

        def idx_at(ref, j):
            return ref.at[lax.rem(lax.div(j, CH), 2), lax.rem(j, CH)]

        pltpu.async_copy(feat_sp.at[idx_at(src_v, 0)], rows4.at[0], gsems[0])
        pltpu.async_copy(feat_sp.at[idx_at(src_v, 1)], rows4.at[1], gsems[1])

        def pair_body(jj, _):
            for p in range(2):
                j = 2 * jj + p
                gs = gsems[p]
                ss = ssems[p]
                b = lax.rem(j, 4)
                bn = lax.rem(j + 2, 4)
                pltpu.make_async_copy(
                    feat_sp.at[idx_at(src_v, j)], rows4.at[b], gs).wait()

                @pl.when(j >= 2)
                def _():
                    pltpu.make_async_copy(
                        rows4.at[bn], acc_sp.at[idx_at(dst_v, j - 2)],
                        ss).wait()

                @pl.when((lax.rem(j + 2, CH) == 0) & (j + 2 < EROWS_PT))
                def _():
                    nsl = lax.rem(lax.div(j + 2, CH), 2)
                    pltpu.make_async_copy(
                        src_hbm.at[s, pl.ds(j + 2, CH)], src_v.at[nsl],
                        isems[0]).wait()
                    pltpu.make_async_copy(
                        dst_hbm.at[s, pl.ds(j + 2, CH)], dst_v.at[nsl],
                        isems[1]).wait()

                @pl.when((lax.rem(j, CH) == 2) & (j - 2 + CH < EROWS_PT)
                         & (j > CH))
                def _():
                    csl = lax.rem(lax.div(j, CH), 2)
                    e2 = j - 2 + CH
                    pltpu.async_copy(
                        src_hbm.at[s, pl.ds(e2, CH)], src_v.at[1 - csl],
                        isems[0])
                    pltpu.async_copy(
                        dst_hbm.at[s, pl.ds(e2, CH)], dst_v.at[1 - csl],
                        isems[1])

                @pl.when(j + 2 < EROWS_PT)
                def _():
                    pltpu.async_copy(
                        feat_sp.at[idx_at(src_v, j + 2)], rows4.at[bn], gs)

                pltpu.async_copy(
                    rows4.at[b], acc_sp.at[idx_at(dst_v, j)], ss, add=True)

                sl = lax.rem(lax.div(j, CH), 2)
                jr = lax.rem(j, CH)
                for t in range(EB // 16):
                    idx = dst_v[sl, jr, pl.ds(16 * t, 16)]
                    plsc.addupdate_scatter(deg_l, [idx], of)
            return 0

        lax.fori_loop(0, EROWS_PT // 2, pair_body, 0)
        for dd in (EROWS_PT - 2, EROWS_PT - 1):
            pltpu.make_async_copy(
                rows4.at[dd % 4], acc_sp.at[idx_at(dst_v, dd)],
                ssems[dd % 2]).wait()

        def merge_body(kk, _):
            def repack(q, _):
                deg_l2[q] = deg_l[pl.ds(512 * kk + 16 * q, 16)]
                return 0

            lax.fori_loop(0, 32, repack, 0)
            pltpu.sync_copy(deg_l2, deg_sp.at[idx64.at[kk]], add=True)
            return 0

        lax.fori_loop(0, 20, merge_body, 0)

        plsc.subcore_barrier()

        pltpu.sync_copy(acc_sp.at[pl.ds(r0, ROWS_PT)],
                        agg_hbm.at[pl.ds(r0, ROWS_PT), pl.ds(c * DH, DH)])

        @pl.when(c == 0)
        def _():
            pltpu.sync_copy(deg_sp.at[pl.ds(s * (DROWS // NS), DROWS // NS)],
                            deg_hbm.at[pl.ds(s * (DROWS // NS), DROWS // NS)])

    return k(feat, src3, dst3)


def _tc1_body(feat_ref, w1_ref, b_ref, o_ref):
    o_ref[...] = jnp.dot(feat_ref[...], w1_ref[...],
                         preferred_element_type=jnp.float32) + b_ref[...]


def _tc2_body(p1_ref, agg_ref, deg_ref, w2_ref, o_ref):
    dinv = 1.0 / jnp.maximum(deg_ref[...], 1.0)
    o_ref[...] = p1_ref[...] + jnp.dot(
        agg_ref[...] * dinv, w2_ref[...], preferred_element_type=jnp.float32)


RB = 1000


def _tc_linear1(feat, w1t, b2d):
    grid = (N_NODES // RB,)
    return pl.pallas_call(
        _tc1_body,
        grid=grid,
        in_specs=[
            pl.BlockSpec((RB, D_IN), lambda i: (i, 0)),
            pl.BlockSpec((D_IN, D_OUT), lambda i: (0, 0)),
            pl.BlockSpec((1, D_OUT), lambda i: (0, 0)),
        ],
        out_specs=pl.BlockSpec((RB, D_OUT), lambda i: (i, 0)),
        out_shape=jax.ShapeDtypeStruct((N_NODES, D_OUT), jnp.float32),
    )(feat, w1t, b2d)


def _tc_linear2(p1, agg, deg, w2t):
    grid = (N_NODES // RB,)
    return pl.pallas_call(
        _tc2_body,
        grid=grid,
        in_specs=[
            pl.BlockSpec((RB, D_OUT), lambda i: (i, 0)),
            pl.BlockSpec((RB, D_IN), lambda i: (i, 0)),
            pl.BlockSpec((RB, 1), lambda i: (i, 0)),
            pl.BlockSpec((D_IN, D_OUT), lambda i: (0, 0)),
        ],
        out_specs=pl.BlockSpec((RB, D_OUT), lambda i: (i, 0)),
        out_shape=jax.ShapeDtypeStruct((N_NODES, D_OUT), jnp.float32),
    )(p1, agg, deg, w2t)


def kernel(feat, edge_index, W, b):
    src3 = edge_index[0].astype(jnp.int32).reshape(NS, EROWS_PT, EB)
    dst3 = edge_index[1].astype(jnp.int32).reshape(NS, EROWS_PT, EB)
    agg, deg = _sc_aggregate(feat, src3, dst3)
    p1 = _tc_linear1(feat, W[:, :D_IN].T, b.reshape(1, D_OUT))
    deg2d = deg.reshape(N_PAD, 1)[:N_NODES]
    return _tc_linear2(p1, agg, deg2d, W[:, D_IN:].T)
